# Initial kernel scaffold; baseline (speedup 1.0000x reference)
import jax
import jax.numpy as jnp
from jax.experimental import pallas as pl


def kernel(x, noise):
    raise NotImplementedError("write your pallas kernel here")


# speedup vs baseline: 1.0166x; 1.0166x over previous
import functools

import jax
import jax.numpy as jnp
from jax import lax
from jax.experimental import pallas as pl
from jax.experimental.pallas import tpu as pltpu
from jax.experimental.pallas import tpu_sc as plsc

NC = 2
NS = 16
NW = NC * NS


def _gather_rows(table, idx_flat):
    R, D = table.shape
    (B,) = idx_flat.shape
    b_per_w = B // NW
    C = 64
    n_chunks = b_per_w // C
    mesh = plsc.VectorSubcoreMesh(core_axis_name="c", subcore_axis_name="s")

    @functools.partial(
        pl.kernel,
        mesh=mesh,
        out_type=jax.ShapeDtypeStruct((B, D), jnp.float32),
        scratch_types=[
            pltpu.VMEM((b_per_w,), jnp.int32),
            pltpu.VMEM((2, C, D), jnp.float32),
            pltpu.SemaphoreType.DMA,
            pltpu.SemaphoreType.DMA,
        ],
    )
    def gk(x_hbm, idx_hbm, out_hbm, idx_v, buf_v, sem0, sem1):
        wid = lax.axis_index("s") * NC + lax.axis_index("c")
        base = wid * b_per_w
        pltpu.sync_copy(idx_hbm.at[pl.ds(base, b_per_w)], idx_v)
        sems = [sem0, sem1]
        cp = pltpu.async_copy(x_hbm.at[idx_v.at[pl.ds(0, C)]], buf_v.at[0], sem0)
        copies = [cp, None]
        for c in range(n_chunks):
            copies[c % 2].wait()
            if c + 1 < n_chunks:
                copies[(c + 1) % 2] = pltpu.async_copy(
                    x_hbm.at[idx_v.at[pl.ds((c + 1) * C, C)]],
                    buf_v.at[(c + 1) % 2],
                    sems[(c + 1) % 2],
                )
            pltpu.sync_copy(buf_v.at[c % 2], out_hbm.at[pl.ds(base + c * C, C)])

    return gk(table, idx_flat)


def kernel(x, noise):
    b, n, d = x.shape
    k = max(1, n // 2)
    _, idx = lax.top_k(noise, k)
    flat_idx = (idx.astype(jnp.int32) + jnp.arange(b, dtype=jnp.int32)[:, None] * n).reshape(-1)
    out = _gather_rows(x.reshape(b * n, d), flat_idx)
    return out.reshape(b, k, d)
